# baseline (speedup 1.0000x reference)
import jax
import jax.numpy as jnp
from jax.experimental import pallas as pl
from jax.experimental.pallas import tpu as pltpu

_N = 32768
_D = 768
_E = 64
_TOP_K = 8
_TEMP = 1.0

_BLOCK = 8192


def _router_body(x_ref, w_ref, scores_ref, weights_ref, idx_ref):
    x = x_ref[...]
    w = w_ref[...]
    s = jax.lax.dot_general(
        x, w, (((1,), (1,)), ((), ())), preferred_element_type=jnp.float32
    )
    scores_ref[...] = s
    st = jax.lax.dot_general(
        w, x, (((1,), (1,)), ((), ())), preferred_element_type=jnp.float32
    )

    expert = jax.lax.broadcasted_iota(jnp.int32, st.shape, 0)
    vals = []
    idxs = []
    for _ in range(_TOP_K):
        top_idx = jnp.argmax(st, axis=0)
        top_val = jnp.max(st, axis=0)
        vals.append(top_val[None, :])
        idxs.append(top_idx[None, :])
        st = jnp.where(expert == top_idx[None, :], -jnp.inf, st)

    top_vals = jnp.concatenate(vals, axis=0)
    top_idxs = jnp.concatenate(idxs, axis=0)
    e = jnp.exp((top_vals - top_vals[0:1, :]) / _TEMP)
    weights_ref[...] = e / jnp.sum(e, axis=0, keepdims=True)
    idx_ref[...] = top_idxs


def kernel(x, W):
    grid = (_N // _BLOCK,)
    scores, weights_t, indices_t = pl.pallas_call(
        _router_body,
        grid=grid,
        in_specs=[
            pl.BlockSpec((_BLOCK, _D), lambda i: (i, 0)),
            pl.BlockSpec((_E, _D), lambda i: (0, 0)),
        ],
        out_specs=[
            pl.BlockSpec((_BLOCK, _E), lambda i: (i, 0)),
            pl.BlockSpec((_TOP_K, _BLOCK), lambda i: (0, i)),
            pl.BlockSpec((_TOP_K, _BLOCK), lambda i: (0, i)),
        ],
        out_shape=[
            jax.ShapeDtypeStruct((_N, _E), jnp.float32),
            jax.ShapeDtypeStruct((_TOP_K, _N), jnp.float32),
            jax.ShapeDtypeStruct((_TOP_K, _N), jnp.int32),
        ],
        compiler_params=pltpu.CompilerParams(
            vmem_limit_bytes=100 * 1024 * 1024
        ),
    )(x, W)
    return (weights_t.T, indices_t.T, scores)


# speedup vs baseline: 1.0258x; 1.0258x over previous
import jax
import jax.numpy as jnp
from jax.experimental import pallas as pl

_N = 32768
_D = 768
_E = 64
_TOP_K = 8
_TEMP = 1.0

_BLOCK = 4096


def _router_body(x_ref, w_ref, scores_ref, weights_ref, idx_ref):
    x = x_ref[...]
    w = w_ref[...]
    s = jax.lax.dot_general(
        x, w, (((1,), (1,)), ((), ())), preferred_element_type=jnp.float32
    )
    scores_ref[...] = s
    st = jax.lax.dot_general(
        w, x, (((1,), (1,)), ((), ())), preferred_element_type=jnp.float32
    )

    expert = jax.lax.broadcasted_iota(jnp.int32, st.shape, 0)
    vals = []
    idxs = []
    for _ in range(_TOP_K):
        top_idx = jnp.argmax(st, axis=0)
        top_val = jnp.max(st, axis=0)
        vals.append(top_val[None, :])
        idxs.append(top_idx[None, :])
        st = jnp.where(expert == top_idx[None, :], -jnp.inf, st)

    top_vals = jnp.concatenate(vals, axis=0)
    top_idxs = jnp.concatenate(idxs, axis=0)
    e = jnp.exp((top_vals - top_vals[0:1, :]) / _TEMP)
    weights_ref[...] = e / jnp.sum(e, axis=0, keepdims=True)
    idx_ref[...] = top_idxs


def kernel(x, W):
    grid = (_N // _BLOCK,)
    scores, weights_t, indices_t = pl.pallas_call(
        _router_body,
        grid=grid,
        in_specs=[
            pl.BlockSpec((_BLOCK, _D), lambda i: (i, 0)),
            pl.BlockSpec((_E, _D), lambda i: (0, 0)),
        ],
        out_specs=[
            pl.BlockSpec((_BLOCK, _E), lambda i: (i, 0)),
            pl.BlockSpec((_TOP_K, _BLOCK), lambda i: (0, i)),
            pl.BlockSpec((_TOP_K, _BLOCK), lambda i: (0, i)),
        ],
        out_shape=[
            jax.ShapeDtypeStruct((_N, _E), jnp.float32),
            jax.ShapeDtypeStruct((_TOP_K, _N), jnp.float32),
            jax.ShapeDtypeStruct((_TOP_K, _N), jnp.int32),
        ],
    )(x, W)
    return (weights_t.T, indices_t.T, scores)
